# Initial kernel scaffold; baseline (speedup 1.0000x reference)
import jax
import jax.numpy as jnp
from jax.experimental import pallas as pl


def kernel(elmo_src, elmo_tgt):
    raise NotImplementedError("write your pallas kernel here")


# speedup vs baseline: 1.2933x; 1.2933x over previous
import jax
import jax.numpy as jnp
from jax.experimental import pallas as pl

_D = 1024
_NL = 3
_B = 16
_LM1 = 511
_L = 512


def _body(in_ref, o0, o1, o2):
    for l, o in enumerate((o0, o1, o2)):
        o[0, 0:1, :] = jnp.zeros((1, _D), jnp.float32)
        o[0, 1:, :] = in_ref[0, :, l * _D:(l + 1) * _D]


def _side(e):
    return pl.pallas_call(
        _body,
        grid=(_B,),
        in_specs=[pl.BlockSpec((1, _LM1, _NL * _D), lambda b: (b, 0, 0))],
        out_specs=[pl.BlockSpec((1, _L, _D), lambda b: (b, 0, 0))] * _NL,
        out_shape=[jax.ShapeDtypeStruct((_B, _L, _D), jnp.float32)] * _NL,
    )(e)


def kernel(elmo_src, elmo_tgt):
    return tuple(_side(elmo_src)) + tuple(_side(elmo_tgt))
